# Initial kernel scaffold; baseline (speedup 1.0000x reference)
import jax
import jax.numpy as jnp
from jax.experimental import pallas as pl


def kernel(x, edge_index, edge_attr, Wp, bp, W1, b1, W2, b2, We, be, gamma, beta):
    raise NotImplementedError("write your pallas kernel here")


# speedup vs baseline: 1.4087x; 1.4087x over previous
import functools

import jax
import jax.numpy as jnp
from jax import lax
from jax.experimental import pallas as pl
from jax.experimental.pallas import tpu as pltpu
from jax.experimental.pallas import tpu_sc as plsc

_N = 10000
_E = 160000
_DIN = 256
_DE = 16
_H = 512
_L = 4

_NSUB = 16
_NCORE = 2
_EB = 128
_EPAD = 163840
_BLOCKS = _EPAD // (_NSUB * _EB)
_ACC_ROWS = 10240
_CHUNK = 128
_NCHUNK = _H // _CHUNK


def _matmul_bias_body(x_ref, w_ref, b_ref, o_ref):
    o_ref[...] = jnp.dot(x_ref[...], w_ref[...],
                         preferred_element_type=jnp.float32) + b_ref[...]


def _matmul_bias(x, w, b, rows_per_block):
    n, k = x.shape
    return pl.pallas_call(
        _matmul_bias_body,
        grid=(n // rows_per_block,),
        in_specs=[
            pl.BlockSpec((rows_per_block, k), lambda i: (i, 0)),
            pl.BlockSpec((k, _H), lambda i: (0, 0)),
            pl.BlockSpec((1, _H), lambda i: (0, 0)),
        ],
        out_specs=pl.BlockSpec((rows_per_block, _H), lambda i: (i, 0)),
        out_shape=jax.ShapeDtypeStruct((n, _H), jnp.float32),
    )(x, w, b.reshape(1, _H))


def _mlp_body(h_ref, a_ref, w1_ref, b1_ref, w2_ref, b2_ref, g_ref, bt_ref,
              hm_ref, ho_ref, hmo_ref):
    h = h_ref[...]
    m = h + a_ref[...]
    t = jnp.maximum(
        jnp.dot(m, w1_ref[...], preferred_element_type=jnp.float32)
        + b1_ref[...], 0.0)
    hn = jnp.dot(t, w2_ref[...], preferred_element_type=jnp.float32) + b2_ref[...]
    u = h + hn
    mu = jnp.mean(u, axis=-1, keepdims=True)
    var = jnp.mean((u - mu) ** 2, axis=-1, keepdims=True)
    y = (u - mu) * lax.rsqrt(var + 1e-5) * g_ref[...] + bt_ref[...]
    y = jnp.maximum(y, 0.0)
    ho_ref[...] = y
    hmo_ref[...] = jnp.maximum(hm_ref[...], y)


def _mlp(h, agg, w1, b1, w2, b2, g, bt, hmax):
    R = 1000
    row = lambda i: (i, 0)
    full = lambda i: (0, 0)
    return pl.pallas_call(
        _mlp_body,
        grid=(_N // R,),
        in_specs=[
            pl.BlockSpec((R, _H), row),
            pl.BlockSpec((R, _H), row),
            pl.BlockSpec((_H, _H), full),
            pl.BlockSpec((1, _H), full),
            pl.BlockSpec((_H, _H), full),
            pl.BlockSpec((1, _H), full),
            pl.BlockSpec((1, _H), full),
            pl.BlockSpec((1, _H), full),
            pl.BlockSpec((R, _H), row),
        ],
        out_specs=[pl.BlockSpec((R, _H), row), pl.BlockSpec((R, _H), row)],
        out_shape=[jax.ShapeDtypeStruct((_N, _H), jnp.float32),
                   jax.ShapeDtypeStruct((_N, _H), jnp.float32)],
    )(h, agg, w1, b1.reshape(1, _H), w2, b2.reshape(1, _H),
      g.reshape(1, _H), bt.reshape(1, _H), hmax)


def _sc_edge(h4, e, srcp, dstp):
    mesh = plsc.VectorSubcoreMesh(core_axis_name="c", subcore_axis_name="s")

    @functools.partial(
        pl.kernel,
        out_type=jax.ShapeDtypeStruct((_N, _H), jnp.float32),
        mesh=mesh,
        scratch_types=[
            pltpu.VMEM((_EB,), jnp.int32),
            pltpu.VMEM((_EB,), jnp.int32),
            pltpu.VMEM((_EB,), jnp.int32),
            pltpu.VMEM((_EB, _CHUNK), jnp.float32),
            pltpu.VMEM((_EB, _CHUNK), jnp.float32),
            pltpu.VMEM_SHARED((_ACC_ROWS, _CHUNK), jnp.float32),
        ],
    )
    def k(h4_hbm, e_hbm, src_hbm, dst_hbm, agg_hbm,
          srcv, idxv, dstv, ev, rowsv, acc):
        cid = lax.axis_index("c")
        sid = lax.axis_index("s")
        epb = _EPAD // _NSUB
        zrows = _ACC_ROWS // _NSUB
        wrows = 624
        for j in range(_NCHUNK // _NCORE):
            c = cid * (_NCHUNK // _NCORE) + j

            @pl.loop(0, _EB)
            def _(r):
                for g in range(_CHUNK // 16):
                    rowsv.at[r][pl.ds(g * 16, 16)] = jnp.zeros((16,), jnp.float32)

            for t in range(zrows // _EB):
                pltpu.sync_copy(rowsv, acc.at[pl.ds(sid * zrows + t * _EB, _EB)])
            plsc.subcore_barrier()

            @pl.loop(0, _BLOCKS)
            def _(b):
                base = sid * epb + b * _EB
                pltpu.sync_copy(src_hbm.at[pl.ds(base, _EB)], srcv)
                pltpu.sync_copy(dst_hbm.at[pl.ds(base, _EB)], dstv)
                for g in range(_EB // 16):
                    s = pl.ds(g * 16, 16)
                    idxv[s] = srcv[s] * _NCHUNK + c
                pltpu.sync_copy(
                    e_hbm.at[pl.ds(base, _EB), pl.ds(c * _CHUNK, _CHUNK)], ev)
                pltpu.sync_copy(h4_hbm.at[idxv], rowsv)

                @pl.loop(0, _EB)
                def _(r):
                    for g in range(_CHUNK // 16):
                        s = pl.ds(g * 16, 16)
                        rowsv.at[r][s] = jnp.maximum(
                            rowsv.at[r][s] + ev.at[r][s], 0.0)

                pltpu.sync_copy(rowsv, acc.at[dstv], add=True)

            plsc.subcore_barrier()
            pltpu.sync_copy(
                acc.at[pl.ds(sid * wrows, wrows)],
                agg_hbm.at[pl.ds(sid * wrows, wrows),
                           pl.ds(c * _CHUNK, _CHUNK)])

            @pl.when(sid == _NSUB - 1)
            def _():
                pltpu.sync_copy(
                    acc.at[pl.ds(_NSUB * wrows, _N - _NSUB * wrows)],
                    agg_hbm.at[pl.ds(_NSUB * wrows, _N - _NSUB * wrows),
                               pl.ds(c * _CHUNK, _CHUNK)])

            plsc.subcore_barrier()

    return k(h4, e, srcp, dstp)


def kernel(x, edge_index, edge_attr, Wp, bp, W1, b1, W2, b2, We, be, gamma, beta):
    src = edge_index[0]
    dst = edge_index[1]
    pad = _EPAD - _E
    srcp = jnp.concatenate([src, jnp.zeros((pad,), jnp.int32)])
    dstp = jnp.concatenate([dst, jnp.full((pad,), _N, jnp.int32)])
    eap = jnp.concatenate([edge_attr, jnp.zeros((pad, _DE), jnp.float32)], axis=0)

    h = _matmul_bias(x, Wp, bp, 1000)
    hmax = jnp.zeros((_N, _H), jnp.float32)
    for l in range(_L):
        e = _matmul_bias(eap, We[l], be[l], 4096)
        agg = _sc_edge(h.reshape(_N * _NCHUNK, _CHUNK), e, srcp, dstp)
        h, hmax = _mlp(h, agg, W1[l], b1[l], W2[l], b2[l],
                       gamma[l], beta[l], hmax)
    return hmax


# speedup vs baseline: 2.1793x; 1.5470x over previous
import functools

import jax
import jax.numpy as jnp
from jax import lax
from jax.experimental import pallas as pl
from jax.experimental.pallas import tpu as pltpu
from jax.experimental.pallas import tpu_sc as plsc

_N = 10000
_E = 160000
_DIN = 256
_DE = 16
_H = 512
_L = 4

_NSUB = 16
_NCORE = 2
_EB = 64
_EPAD = 163840
_BLOCKS = _EPAD // (_NSUB * _EB)
_ACC_ROWS = 10048
_CHUNK = 128
_NCHUNK = _H // _CHUNK


def _matmul_bias_body(x_ref, w_ref, b_ref, o_ref):
    o_ref[...] = jnp.dot(x_ref[...], w_ref[...],
                         preferred_element_type=jnp.float32) + b_ref[...]


def _matmul_bias(x, w, b, rows_per_block):
    n, k = x.shape
    return pl.pallas_call(
        _matmul_bias_body,
        grid=(n // rows_per_block,),
        in_specs=[
            pl.BlockSpec((rows_per_block, k), lambda i: (i, 0)),
            pl.BlockSpec((k, _H), lambda i: (0, 0)),
            pl.BlockSpec((1, _H), lambda i: (0, 0)),
        ],
        out_specs=pl.BlockSpec((rows_per_block, _H), lambda i: (i, 0)),
        out_shape=jax.ShapeDtypeStruct((n, _H), jnp.float32),
    )(x, w, b.reshape(1, _H))


def _mlp_body(h_ref, a_ref, w1_ref, b1_ref, w2_ref, b2_ref, g_ref, bt_ref,
              hm_ref, ho_ref, hmo_ref):
    h = h_ref[...]
    m = h + a_ref[...]
    t = jnp.maximum(
        jnp.dot(m, w1_ref[...], preferred_element_type=jnp.float32)
        + b1_ref[...], 0.0)
    hn = jnp.dot(t, w2_ref[...], preferred_element_type=jnp.float32) + b2_ref[...]
    u = h + hn
    mu = jnp.mean(u, axis=-1, keepdims=True)
    var = jnp.mean((u - mu) ** 2, axis=-1, keepdims=True)
    y = (u - mu) * lax.rsqrt(var + 1e-5) * g_ref[...] + bt_ref[...]
    y = jnp.maximum(y, 0.0)
    ho_ref[...] = y
    hmo_ref[...] = jnp.maximum(hm_ref[...], y)


def _mlp(h, agg, w1, b1, w2, b2, g, bt, hmax):
    R = 1000
    row = lambda i: (i, 0)
    full = lambda i: (0, 0)
    return pl.pallas_call(
        _mlp_body,
        grid=(_N // R,),
        in_specs=[
            pl.BlockSpec((R, _H), row),
            pl.BlockSpec((R, _H), row),
            pl.BlockSpec((_H, _H), full),
            pl.BlockSpec((1, _H), full),
            pl.BlockSpec((_H, _H), full),
            pl.BlockSpec((1, _H), full),
            pl.BlockSpec((1, _H), full),
            pl.BlockSpec((1, _H), full),
            pl.BlockSpec((R, _H), row),
        ],
        out_specs=[pl.BlockSpec((R, _H), row), pl.BlockSpec((R, _H), row)],
        out_shape=[jax.ShapeDtypeStruct((_N, _H), jnp.float32),
                   jax.ShapeDtypeStruct((_N, _H), jnp.float32)],
    )(h, agg, w1, b1.reshape(1, _H), w2, b2.reshape(1, _H),
      g.reshape(1, _H), bt.reshape(1, _H), hmax)


def _sc_edge(h4, e, srcp, dstp):
    mesh = plsc.VectorSubcoreMesh(core_axis_name="c", subcore_axis_name="s")

    @functools.partial(
        pl.kernel,
        out_type=jax.ShapeDtypeStruct((_N, _H), jnp.float32),
        mesh=mesh,
        scratch_types=(
            [pltpu.VMEM((_EB,), jnp.int32) for _ in range(2)]
            + [pltpu.VMEM((_EB,), jnp.int32) for _ in range(2)]
            + [pltpu.VMEM((_EB,), jnp.int32) for _ in range(2)]
            + [pltpu.VMEM((_EB,), jnp.int32) for _ in range(4)]
            + [pltpu.VMEM((_EB, _CHUNK), jnp.float32) for _ in range(2)]
            + [pltpu.VMEM((_EB, _CHUNK), jnp.float32) for _ in range(4)]
            + [pltpu.VMEM_SHARED((_ACC_ROWS, _CHUNK), jnp.float32)]
            + [pltpu.SemaphoreType.DMA for _ in range(8)]
        ),
    )
    def k(h4_hbm, e_hbm, src_hbm, dst_hbm, agg_hbm,
          s0, s1, d0, d1, x0, x1, q0, q1, q2, q3, e0, e1,
          r0, r1, r2, r3, acc, si0, si1, sg0, sg1, ss0, ss1, ss2, ss3):
        srcv = [s0, s1]
        dstv = [d0, d1]
        idxv = [x0, x1]
        scix = [q0, q1, q2, q3]
        ev = [e0, e1]
        rows = [r0, r1, r2, r3]
        sem_i = [si0, si1]
        sem_g = [sg0, sg1]
        sem_s = [ss0, ss1, ss2, ss3]

        cid = lax.axis_index("c")
        sid = lax.axis_index("s")
        epb = _EPAD // _NSUB
        wrows = 624

        def idx_start(bb, p):
            base = sid * epb + bb * _EB
            pltpu.make_async_copy(
                src_hbm.at[pl.ds(base, _EB)], srcv[p], sem_i[p]).start()
            pltpu.make_async_copy(
                dst_hbm.at[pl.ds(base, _EB)], dstv[p], sem_i[p]).start()

        def idx_wait(p):
            pltpu.make_async_copy(
                src_hbm.at[pl.ds(0, _EB)], srcv[p], sem_i[p]).wait()
            pltpu.make_async_copy(
                dst_hbm.at[pl.ds(0, _EB)], dstv[p], sem_i[p]).wait()

        def ge_start(bb, p, r, c):
            base = sid * epb + bb * _EB
            pltpu.make_async_copy(
                e_hbm.at[pl.ds(base, _EB), pl.ds(c * _CHUNK, _CHUNK)],
                ev[p], sem_g[p]).start()
            pltpu.make_async_copy(
                h4_hbm.at[idxv[p]], rows[r], sem_g[p]).start()

        def ge_wait(p, r, c):
            pltpu.make_async_copy(
                e_hbm.at[pl.ds(0, _EB), pl.ds(c * _CHUNK, _CHUNK)],
                ev[p], sem_g[p]).wait()
            pltpu.make_async_copy(
                h4_hbm.at[idxv[p]], rows[r], sem_g[p]).wait()

        def sc_wait(r):
            pltpu.make_async_copy(rows[r], acc.at[scix[r]], sem_s[r]).wait()

        for j in range(_NCHUNK // _NCORE):
            c = cid * (_NCHUNK // _NCORE) + j

            @pl.loop(0, _EB)
            def _(r):
                for g in range(_CHUNK // 16):
                    rows[0].at[r][pl.ds(g * 16, 16)] = jnp.zeros(
                        (16,), jnp.float32)

            ztiles = _ACC_ROWS // _EB
            for t in range(-(-ztiles // _NSUB)):
                tile = sid + t * _NSUB

                @pl.when(tile < ztiles)
                def _():
                    pltpu.sync_copy(rows[0], acc.at[pl.ds(tile * _EB, _EB)])
            plsc.subcore_barrier()

            idx_start(0, 0)
            idx_start(1, 1)

            @pl.loop(0, _BLOCKS // 4)
            def _(g):
                for u in range(4):
                    p = u % 2
                    q = 1 - p
                    rp = (u - 1) % 4
                    bb = g * 4 + u
                    idx_wait(p)

                    @pl.when(g >= 1)
                    def _():
                        sc_wait(u)

                    for t in range(_EB // 16):
                        s = pl.ds(t * 16, 16)
                        idxv[p][s] = srcv[p][s] * _NCHUNK + c
                        scix[u][s] = dstv[p][s]
                    ge_start(bb, p, u, c)
                    if u < 2:
                        idx_start(bb + 2, p)
                    else:
                        @pl.when(g < _BLOCKS // 4 - 1)
                        def _():
                            idx_start(bb + 2, p)

                    def compute_and_scatter(qq, rr):
                        ge_wait(qq, rr, c)

                        @pl.loop(0, _EB)
                        def _(row):
                            for t in range(_CHUNK // 16):
                                s = pl.ds(t * 16, 16)
                                rows[rr].at[row][s] = jnp.maximum(
                                    rows[rr].at[row][s] + ev[qq].at[row][s],
                                    0.0)

                        pltpu.async_copy(rows[rr], acc.at[scix[rr]],
                                         sem_s[rr], add=True)

                    if u == 0:
                        @pl.when(g >= 1)
                        def _():
                            compute_and_scatter(q, rp)
                    else:
                        compute_and_scatter(q, rp)

            compute_done_p = (_BLOCKS - 1) % 2
            compute_done_r = (_BLOCKS - 1) % 4

            def final_cs():
                ge_wait(compute_done_p, compute_done_r, c)

                @pl.loop(0, _EB)
                def _(row):
                    for t in range(_CHUNK // 16):
                        s = pl.ds(t * 16, 16)
                        rows[compute_done_r].at[row][s] = jnp.maximum(
                            rows[compute_done_r].at[row][s]
                            + ev[compute_done_p].at[row][s], 0.0)

                pltpu.async_copy(rows[compute_done_r],
                                 acc.at[scix[compute_done_r]],
                                 sem_s[compute_done_r], add=True)

            final_cs()
            for r in range(4):
                sc_wait(r)

            plsc.subcore_barrier()
            pltpu.sync_copy(
                acc.at[pl.ds(sid * wrows, wrows)],
                agg_hbm.at[pl.ds(sid * wrows, wrows),
                           pl.ds(c * _CHUNK, _CHUNK)])

            @pl.when(sid == _NSUB - 1)
            def _():
                pltpu.sync_copy(
                    acc.at[pl.ds(_NSUB * wrows, _N - _NSUB * wrows)],
                    agg_hbm.at[pl.ds(_NSUB * wrows, _N - _NSUB * wrows),
                               pl.ds(c * _CHUNK, _CHUNK)])

            plsc.subcore_barrier()

    return k(h4, e, srcp, dstp)


def kernel(x, edge_index, edge_attr, Wp, bp, W1, b1, W2, b2, We, be, gamma, beta):
    src = edge_index[0]
    dst = edge_index[1]
    pad = _EPAD - _E
    srcp = jnp.concatenate([src, jnp.zeros((pad,), jnp.int32)])
    dstp = jnp.concatenate([dst, jnp.full((pad,), _N, jnp.int32)])
    eap = jnp.concatenate([edge_attr, jnp.zeros((pad, _DE), jnp.float32)], axis=0)

    h = _matmul_bias(x, Wp, bp, 1000)
    hmax = jnp.zeros((_N, _H), jnp.float32)
    for l in range(_L):
        e = _matmul_bias(eap, We[l], be[l], 4096)
        agg = _sc_edge(h.reshape(_N * _NCHUNK, _CHUNK), e, srcp, dstp)
        h, hmax = _mlp(h, agg, W1[l], b1[l], W2[l], b2[l],
                       gamma[l], beta[l], hmax)
    return hmax
